# Initial kernel scaffold; baseline (speedup 1.0000x reference)
import jax
import jax.numpy as jnp
from jax.experimental import pallas as pl


def kernel(user_emb, item_emb, edge_index, edge_weight, users, items):
    raise NotImplementedError("write your pallas kernel here")


# speedup vs baseline: 3.2288x; 3.2288x over previous
import functools

import jax
import jax.numpy as jnp
from jax import lax
from jax.experimental import pallas as pl
from jax.experimental.pallas import tpu as pltpu
from jax.experimental.pallas import tpu_sc as plsc

N_NODES = 10000
N_PAD = 10240
N_EDGES = 320000
D = 128
N_USERS = 5000
BATCH_N = 4096

NC = 2
NS = 16
NW = NC * NS
EPW = N_EDGES // NW
CHUNK = 80
NCHUNK = EPW // CHUNK
ROWS_PT = N_PAD // NS
ZROWS = 128

FLAT = N_PAD * D
FPW = FLAT // NW
FSUB = 8192
BPW = BATCH_N // NW

_mesh = plsc.VectorSubcoreMesh(core_axis_name="c", subcore_axis_name="s")


def _wid():
    return lax.axis_index("c") * NS + lax.axis_index("s")


_GDN = lax.GatherDimensionNumbers(
    offset_dims=(), collapsed_slice_dims=(0,), start_index_map=(0,))


def _lane_perm(vec16, idx16):
    return lax.gather(vec16, idx16.reshape(16, 1).astype(jnp.int32), _GDN,
                      (1,), mode=lax.GatherScatterMode.PROMISE_IN_BOUNDS)


def _lane_bcast(vec16, l):
    return _lane_perm(vec16, jnp.full((16,), l, jnp.int32))


def _lane_sum(vec16):
    lane = lax.iota(jnp.int32, 16)
    for sh in (1, 2, 4, 8):
        vec16 = vec16 + _lane_perm(vec16, lane ^ sh)
    return vec16


@functools.partial(
    pl.kernel,
    out_type=(
        jax.ShapeDtypeStruct((N_PAD, D), jnp.float32),
        jax.ShapeDtypeStruct((N_PAD, D), jnp.float32),
    ),
    mesh=_mesh,
    scratch_types=[
        pltpu.VMEM_SHARED((N_PAD, D), jnp.float32),
        pltpu.VMEM((CHUNK,), jnp.int32),
        pltpu.VMEM((CHUNK,), jnp.int32),
        pltpu.VMEM((CHUNK,), jnp.float32),
        pltpu.VMEM((CHUNK, D), jnp.float32),
        pltpu.VMEM((ZROWS, D), jnp.float32),
        pltpu.SemaphoreType.DMA,
    ],
)
def _prop(h_hbm, src_hbm, dst_hbm, w_hbm, out0, out1,
          acc, src_v, dst_v, w_v, rows_v, zero_v, sem):
    c = lax.axis_index("c")
    s = lax.axis_index("s")
    wid = c * NS + s

    zvec = jnp.zeros((16,), jnp.float32)

    def _zrow(i, _):
        for v in range(D // 16):
            zero_v[i, pl.ds(16 * v, 16)] = zvec
        return 0

    lax.fori_loop(0, ZROWS, _zrow, 0)
    for r in range(ROWS_PT // ZROWS):
        pltpu.sync_copy(zero_v, acc.at[pl.ds(s * ROWS_PT + r * ZROWS, ZROWS)])
    plsc.subcore_barrier()

    ebase = wid * EPW

    def _chunk(ci, _):
        off = ebase + ci * CHUNK
        pltpu.sync_copy(src_hbm.at[pl.ds(off, CHUNK)], src_v)
        pltpu.sync_copy(dst_hbm.at[pl.ds(off, CHUNK)], dst_v)
        pltpu.sync_copy(w_hbm.at[pl.ds(off, CHUNK)], w_v)
        pltpu.async_copy(h_hbm.at[src_v], rows_v, sem).wait()

        def _scale(j, _):
            w16 = w_v[pl.ds((j // 16) * 16, 16)]
            wb = _lane_bcast(w16, j % 16)
            for v in range(D // 16):
                rows_v[j, pl.ds(16 * v, 16)] = (
                    rows_v[j, pl.ds(16 * v, 16)] * wb)
            return 0

        lax.fori_loop(0, CHUNK, _scale, 0)
        pltpu.sync_copy(rows_v, acc.at[dst_v], add=True)
        return 0

    lax.fori_loop(0, NCHUNK, _chunk, 0)
    plsc.subcore_barrier()

    @pl.when(c == 0)
    def _():
        pltpu.sync_copy(acc.at[pl.ds(s * ROWS_PT, ROWS_PT)],
                        out0.at[pl.ds(s * ROWS_PT, ROWS_PT)])

    @pl.when(c == 1)
    def _():
        pltpu.sync_copy(acc.at[pl.ds(s * ROWS_PT, ROWS_PT)],
                        out1.at[pl.ds(s * ROWS_PT, ROWS_PT)])


@functools.partial(
    pl.kernel,
    out_type=(
        jax.ShapeDtypeStruct((FLAT,), jnp.float32),
        jax.ShapeDtypeStruct((FLAT,), jnp.float32),
    ),
    mesh=_mesh,
    scratch_types=[
        pltpu.VMEM((FSUB,), jnp.float32),
        pltpu.VMEM((FSUB,), jnp.float32),
        pltpu.VMEM((FSUB,), jnp.float32),
    ],
)
def _combine(p0, p1, sum_in, h_out, sum_out, a_v, b_v, s_v):
    wid = _wid()
    for k in range(FPW // FSUB):
        off = wid * FPW + k * FSUB
        pltpu.sync_copy(p0.at[pl.ds(off, FSUB)], a_v)
        pltpu.sync_copy(p1.at[pl.ds(off, FSUB)], b_v)
        pltpu.sync_copy(sum_in.at[pl.ds(off, FSUB)], s_v)

        def _add(i, _):
            sl = pl.ds(i * 16, 16)
            h = a_v[sl] + b_v[sl]
            a_v[sl] = h
            s_v[sl] = s_v[sl] + h
            return 0

        lax.fori_loop(0, FSUB // 16, _add, 0)
        pltpu.sync_copy(a_v, h_out.at[pl.ds(off, FSUB)])
        pltpu.sync_copy(s_v, sum_out.at[pl.ds(off, FSUB)])


@functools.partial(
    pl.kernel,
    out_type=jax.ShapeDtypeStruct((BATCH_N,), jnp.float32),
    mesh=_mesh,
    scratch_types=[
        pltpu.VMEM((BPW,), jnp.int32),
        pltpu.VMEM((BPW,), jnp.int32),
        pltpu.VMEM((BPW,), jnp.int32),
        pltpu.VMEM((BPW, D), jnp.float32),
        pltpu.VMEM((BPW, D), jnp.float32),
        pltpu.VMEM((BPW,), jnp.float32),
        pltpu.SemaphoreType.DMA,
    ],
)
def _gamma(sum_hbm, users_hbm, items_hbm, out_hbm,
           u_v, it_v, ii_v, ur_v, ir_v, out_v, sem):
    wid = _wid()
    off = wid * BPW
    pltpu.sync_copy(users_hbm.at[pl.ds(off, BPW)], u_v)
    pltpu.sync_copy(items_hbm.at[pl.ds(off, BPW)], it_v)

    def _shift(i, _):
        sl = pl.ds(i * 16, 16)
        ii_v[sl] = it_v[sl] + N_USERS
        return 0

    lax.fori_loop(0, BPW // 16, _shift, 0)
    pltpu.async_copy(sum_hbm.at[u_v], ur_v, sem).wait()
    pltpu.async_copy(sum_hbm.at[ii_v], ir_v, sem).wait()

    lane = lax.iota(jnp.int32, 16)

    def _group(g, _):
        def _one(l, accv):
            b = g * 16 + l
            acc = jnp.zeros((16,), jnp.float32)
            for v in range(D // 16):
                sl = pl.ds(16 * v, 16)
                acc = acc + ur_v[b, sl] * ir_v[b, sl]
            gvec = _lane_sum(acc) * (1.0 / 16.0)
            return jnp.where(lane == l, gvec, accv)

        vec = lax.fori_loop(0, 16, _one, jnp.zeros((16,), jnp.float32))
        out_v[pl.ds(g * 16, 16)] = vec
        return 0

    lax.fori_loop(0, BPW // 16, _group, 0)
    pltpu.sync_copy(out_v, out_hbm.at[pl.ds(off, BPW)])


def kernel(user_emb, item_emb, edge_index, edge_weight, users, items):
    all_emb = jnp.concatenate([user_emb, item_emb], axis=0)
    src = edge_index[0].astype(jnp.int32)
    dst = edge_index[1].astype(jnp.int32)
    users = users.astype(jnp.int32)
    items = items.astype(jnp.int32)

    pad = jnp.zeros((N_PAD - N_NODES, D), jnp.float32)
    h = jnp.concatenate([all_emb, pad], axis=0)
    ssum = h.reshape(FLAT)
    for _ in range(3):
        p0, p1 = _prop(h, src, dst, edge_weight)
        h_flat, ssum = _combine(p0.reshape(FLAT), p1.reshape(FLAT), ssum)
        h = h_flat.reshape(N_PAD, D)
    return _gamma(ssum.reshape(N_PAD, D), users, items)


# speedup vs baseline: 9.8312x; 3.0448x over previous
import functools

import jax
import jax.numpy as jnp
from jax import lax
from jax.experimental import pallas as pl
from jax.experimental.pallas import tpu as pltpu
from jax.experimental.pallas import tpu_sc as plsc

N_NODES = 10000
N_PAD = 10240
N_EDGES = 320000
D = 128
N_USERS = 5000
BATCH_N = 4096

NC = 2
NS = 16
NW = NC * NS
EPW = N_EDGES // NW
CHUNK = 80
NCHUNK = EPW // CHUNK
NPAIR = NCHUNK // 2
ROWS_PT = N_PAD // NS
ZROWS = 128

BPW = BATCH_N // NW

_mesh = plsc.VectorSubcoreMesh(core_axis_name="c", subcore_axis_name="s")

_GDN = lax.GatherDimensionNumbers(
    offset_dims=(), collapsed_slice_dims=(0,), start_index_map=(0,))


def _lane_perm(vec16, idx16):
    return lax.gather(vec16, idx16.reshape(16, 1).astype(jnp.int32), _GDN,
                      (1,), mode=lax.GatherScatterMode.PROMISE_IN_BOUNDS)


def _lane_bcast(vec16, l):
    return _lane_perm(vec16, jnp.full((16,), l, jnp.int32))


def _lane_sum(vec16):
    lane = lax.iota(jnp.int32, 16)
    for sh in (1, 2, 4, 8):
        vec16 = vec16 + _lane_perm(vec16, lane ^ sh)
    return vec16


@functools.partial(
    pl.kernel,
    out_type=(
        jax.ShapeDtypeStruct((N_PAD, D), jnp.float32),
        jax.ShapeDtypeStruct((N_PAD, D), jnp.float32),
    ),
    mesh=_mesh,
    scratch_types=[
        pltpu.VMEM_SHARED((N_PAD, D), jnp.float32),
        pltpu.VMEM((EPW,), jnp.int32),
        pltpu.VMEM((EPW,), jnp.float32),
        pltpu.VMEM((CHUNK,), jnp.int32),
        pltpu.VMEM((CHUNK,), jnp.int32),
        pltpu.VMEM((CHUNK,), jnp.int32),
        pltpu.VMEM((CHUNK,), jnp.int32),
        pltpu.VMEM((CHUNK, D), jnp.float32),
        pltpu.VMEM((CHUNK, D), jnp.float32),
        pltpu.SemaphoreType.DMA,
        pltpu.SemaphoreType.DMA,
        pltpu.SemaphoreType.DMA,
        pltpu.SemaphoreType.DMA,
    ],
)
def _prop(h_hbm, src_hbm, dst_hbm, w_hbm, out0, out1,
          acc, srcs_v, ws_v, src_a, src_b, dst_a, dst_b,
          rows_a, rows_b, sem_a, sem_b, sem_da, sem_db):
    c = lax.axis_index("c")
    s = lax.axis_index("s")
    wid = c * NS + s
    ebase = wid * EPW

    pltpu.sync_copy(src_hbm.at[pl.ds(ebase, EPW)], srcs_v)
    pltpu.sync_copy(w_hbm.at[pl.ds(ebase, EPW)], ws_v)

    zvec = jnp.zeros((16,), jnp.float32)

    def _zrow(i, _):
        for v in range(D // 16):
            rows_a[i, pl.ds(16 * v, 16)] = zvec
        return 0

    lax.fori_loop(0, CHUNK, _zrow, 0)
    for r in range(ROWS_PT // CHUNK):
        pltpu.sync_copy(rows_a, acc.at[pl.ds(s * ROWS_PT + r * CHUNK, CHUNK)])
    plsc.subcore_barrier()

    def _fetch_dst(ci, dbuf, sem):
        pltpu.async_copy(
            dst_hbm.at[pl.ds(ebase + ci * CHUNK, CHUNK)], dbuf, sem)

    def _issue(ci, sbuf, rbuf, sem):
        for g in range(CHUNK // 16):
            sbuf[pl.ds(g * 16, 16)] = srcs_v[pl.ds(ci * CHUNK + g * 16, 16)]
        pltpu.async_copy(h_hbm.at[sbuf], rbuf, sem)

    def _wait_rows(rbuf, sem):
        pltpu.make_async_copy(h_hbm.at[pl.ds(0, CHUNK)], rbuf, sem).wait()

    def _wait_dst(dbuf, sem):
        pltpu.make_async_copy(dst_hbm.at[pl.ds(0, CHUNK)], dbuf, sem).wait()

    def _scale_scatter(ci, rbuf, dbuf):
        def _grp(g, _):
            w16 = ws_v[pl.ds(ci * CHUNK + g * 16, 16)]
            for l in range(16):
                wb = _lane_bcast(w16, l)
                j = g * 16 + l
                for v in range(D // 16):
                    rsl = pl.ds(16 * v, 16)
                    rbuf[j, rsl] = rbuf[j, rsl] * wb
            return 0

        lax.fori_loop(0, CHUNK // 16, _grp, 0)
        pltpu.sync_copy(rbuf, acc.at[dbuf], add=True)

    _fetch_dst(0, dst_a, sem_da)
    _issue(0, src_a, rows_a, sem_a)

    def _pair(p, _):
        ci = 2 * p
        _fetch_dst(ci + 1, dst_b, sem_db)
        _issue(ci + 1, src_b, rows_b, sem_b)
        _wait_rows(rows_a, sem_a)
        _wait_dst(dst_a, sem_da)
        _scale_scatter(ci, rows_a, dst_a)

        @pl.when(ci + 2 < NCHUNK)
        def _():
            _fetch_dst(ci + 2, dst_a, sem_da)
            _issue(ci + 2, src_a, rows_a, sem_a)

        _wait_rows(rows_b, sem_b)
        _wait_dst(dst_b, sem_db)
        _scale_scatter(ci + 1, rows_b, dst_b)
        return 0

    lax.fori_loop(0, NPAIR, _pair, 0)
    _wait_rows(rows_a, sem_a)
    _wait_dst(dst_a, sem_da)
    _scale_scatter(NCHUNK - 1, rows_a, dst_a)
    plsc.subcore_barrier()

    @pl.when(c == 0)
    def _():
        pltpu.sync_copy(acc.at[pl.ds(s * ROWS_PT, ROWS_PT)],
                        out0.at[pl.ds(s * ROWS_PT, ROWS_PT)])

    @pl.when(c == 1)
    def _():
        pltpu.sync_copy(acc.at[pl.ds(s * ROWS_PT, ROWS_PT)],
                        out1.at[pl.ds(s * ROWS_PT, ROWS_PT)])


def _combine_body(p0_ref, p1_ref, s_ref, h_ref, so_ref):
    h = p0_ref[...] + p1_ref[...]
    h_ref[...] = h
    so_ref[...] = s_ref[...] + h


_CROWS = 1024


def _combine(p0, p1, s_in):
    spec = pl.BlockSpec((_CROWS, D), lambda i: (i, 0))
    return pl.pallas_call(
        _combine_body,
        grid=(N_PAD // _CROWS,),
        in_specs=[spec, spec, spec],
        out_specs=[spec, spec],
        out_shape=[
            jax.ShapeDtypeStruct((N_PAD, D), jnp.float32),
            jax.ShapeDtypeStruct((N_PAD, D), jnp.float32),
        ],
    )(p0, p1, s_in)


@functools.partial(
    pl.kernel,
    out_type=jax.ShapeDtypeStruct((BATCH_N,), jnp.float32),
    mesh=_mesh,
    scratch_types=[
        pltpu.VMEM((BPW,), jnp.int32),
        pltpu.VMEM((BPW,), jnp.int32),
        pltpu.VMEM((BPW,), jnp.int32),
        pltpu.VMEM((BPW, D), jnp.float32),
        pltpu.VMEM((BPW, D), jnp.float32),
        pltpu.VMEM((BPW,), jnp.float32),
        pltpu.SemaphoreType.DMA,
    ],
)
def _gamma(sum_hbm, users_hbm, items_hbm, out_hbm,
           u_v, it_v, ii_v, ur_v, ir_v, out_v, sem):
    wid = lax.axis_index("c") * NS + lax.axis_index("s")
    off = wid * BPW
    pltpu.sync_copy(users_hbm.at[pl.ds(off, BPW)], u_v)
    pltpu.sync_copy(items_hbm.at[pl.ds(off, BPW)], it_v)

    def _shift(i, _):
        sl = pl.ds(i * 16, 16)
        ii_v[sl] = it_v[sl] + N_USERS
        return 0

    lax.fori_loop(0, BPW // 16, _shift, 0)
    pltpu.async_copy(sum_hbm.at[u_v], ur_v, sem).wait()
    pltpu.async_copy(sum_hbm.at[ii_v], ir_v, sem).wait()

    lane = lax.iota(jnp.int32, 16)

    def _group(g, _):
        def _one(l, accv):
            b = g * 16 + l
            acc = jnp.zeros((16,), jnp.float32)
            for v in range(D // 16):
                sl = pl.ds(16 * v, 16)
                acc = acc + ur_v[b, sl] * ir_v[b, sl]
            gvec = _lane_sum(acc) * (1.0 / 16.0)
            return jnp.where(lane == l, gvec, accv)

        vec = lax.fori_loop(0, 16, _one, jnp.zeros((16,), jnp.float32))
        out_v[pl.ds(g * 16, 16)] = vec
        return 0

    lax.fori_loop(0, BPW // 16, _group, 0)
    pltpu.sync_copy(out_v, out_hbm.at[pl.ds(off, BPW)])


def kernel(user_emb, item_emb, edge_index, edge_weight, users, items):
    all_emb = jnp.concatenate([user_emb, item_emb], axis=0)
    src = edge_index[0].astype(jnp.int32)
    dst = edge_index[1].astype(jnp.int32)
    users = users.astype(jnp.int32)
    items = items.astype(jnp.int32)

    pad = jnp.zeros((N_PAD - N_NODES, D), jnp.float32)
    h = jnp.concatenate([all_emb, pad], axis=0)
    ssum = h
    for _ in range(3):
        p0, p1 = _prop(h, src, dst, edge_weight)
        h, ssum = _combine(p0, p1, ssum)
    return _gamma(ssum, users, items)


# speedup vs baseline: 10.6909x; 1.0874x over previous
import functools

import jax
import jax.numpy as jnp
from jax import lax
from jax.experimental import pallas as pl
from jax.experimental.pallas import tpu as pltpu
from jax.experimental.pallas import tpu_sc as plsc

N_NODES = 10000
N_PAD = 10240
N_EDGES = 320000
D = 128
N_USERS = 5000
BATCH_N = 4096

NC = 2
NS = 16
NW = NC * NS
EPW = N_EDGES // NW
CHUNK = 64
NFULL = EPW // CHUNK
TAIL = EPW - NFULL * CHUNK
ROWS_PT = N_PAD // NS

BPW = BATCH_N // NW

_mesh = plsc.VectorSubcoreMesh(core_axis_name="c", subcore_axis_name="s")

_GDN = lax.GatherDimensionNumbers(
    offset_dims=(), collapsed_slice_dims=(0,), start_index_map=(0,))


def _lane_perm(vec16, idx16):
    return lax.gather(vec16, idx16.reshape(16, 1).astype(jnp.int32), _GDN,
                      (1,), mode=lax.GatherScatterMode.PROMISE_IN_BOUNDS)


def _lane_bcast(vec16, l):
    return _lane_perm(vec16, jnp.full((16,), l, jnp.int32))


def _lane_sum(vec16):
    lane = lax.iota(jnp.int32, 16)
    for sh in (1, 2, 4, 8):
        vec16 = vec16 + _lane_perm(vec16, lane ^ sh)
    return vec16


@functools.partial(
    pl.kernel,
    out_type=(
        jax.ShapeDtypeStruct((N_PAD, D), jnp.float32),
        jax.ShapeDtypeStruct((N_PAD, D), jnp.float32),
    ),
    mesh=_mesh,
    scratch_types=[
        pltpu.VMEM_SHARED((N_PAD, D), jnp.float32),
        pltpu.VMEM((EPW,), jnp.int32),
        pltpu.VMEM((EPW,), jnp.float32),
        pltpu.VMEM((CHUNK, D), jnp.float32),
        pltpu.VMEM((CHUNK, D), jnp.float32),
        pltpu.VMEM((CHUNK, D), jnp.float32),
        pltpu.VMEM((CHUNK,), jnp.int32),
        pltpu.VMEM((CHUNK,), jnp.int32),
        pltpu.VMEM((CHUNK,), jnp.int32),
        pltpu.VMEM((TAIL, D), jnp.float32),
        pltpu.VMEM((TAIL,), jnp.int32),
        pltpu.SemaphoreType.DMA,
        pltpu.SemaphoreType.DMA,
        pltpu.SemaphoreType.DMA,
        pltpu.SemaphoreType.DMA,
        pltpu.SemaphoreType.DMA,
        pltpu.SemaphoreType.DMA,
        pltpu.SemaphoreType.DMA,
        pltpu.SemaphoreType.DMA,
        pltpu.SemaphoreType.DMA,
    ],
)
def _prop(h_hbm, src_hbm, dst_hbm, w_hbm, out0, out1,
          acc, srcs_v, ws_v, rows0, rows1, rows2, dst0, dst1, dst2,
          rows_t, dst_t,
          sg0, sg1, sg2, sd0, sd1, sd2, sw0, sw1, sw2):
    c = lax.axis_index("c")
    s = lax.axis_index("s")
    wid = c * NS + s
    ebase = wid * EPW

    rows = (rows0, rows1, rows2)
    dsts = (dst0, dst1, dst2)
    sgs = (sg0, sg1, sg2)
    sds = (sd0, sd1, sd2)
    sws = (sw0, sw1, sw2)

    pltpu.sync_copy(src_hbm.at[pl.ds(ebase, EPW)], srcs_v)
    pltpu.sync_copy(w_hbm.at[pl.ds(ebase, EPW)], ws_v)

    zvec = jnp.zeros((16,), jnp.float32)

    def _zrow(i, _):
        for v in range(D // 16):
            rows0[i, pl.ds(16 * v, 16)] = zvec
        return 0

    lax.fori_loop(0, CHUNK, _zrow, 0)
    for r in range(ROWS_PT // CHUNK):
        pltpu.sync_copy(rows0, acc.at[pl.ds(s * ROWS_PT + r * CHUNK, CHUNK)])
    plsc.subcore_barrier()

    def _fetch(ci, x):
        pltpu.async_copy(
            dst_hbm.at[pl.ds(ebase + ci * CHUNK, CHUNK)], dsts[x], sds[x])

    def _gissue(ci, x):
        pltpu.async_copy(
            h_hbm.at[srcs_v.at[pl.ds(ci * CHUNK, CHUNK)]], rows[x], sgs[x])

    def _gwait(x):
        pltpu.make_async_copy(h_hbm.at[pl.ds(0, CHUNK)], rows[x], sgs[x]).wait()
        pltpu.make_async_copy(dst_hbm.at[pl.ds(0, CHUNK)], dsts[x],
                              sds[x]).wait()

    def _wissue(x):
        pltpu.async_copy(rows[x], acc.at[dsts[x]], sws[x], add=True)

    def _wwait(x):
        pltpu.make_async_copy(rows[x], acc.at[dsts[x]], sws[x]).wait()

    def _scale(ci, rbuf, nedge):
        def _grp(g, _):
            w16 = ws_v[pl.ds(ci * CHUNK + g * 16, 16)]
            for l in range(16):
                wb = _lane_bcast(w16, l)
                j = g * 16 + l
                for v in range(D // 16):
                    rsl = pl.ds(16 * v, 16)
                    rbuf[j, rsl] = rbuf[j, rsl] * wb
            return 0

        lax.fori_loop(0, nedge // 16, _grp, 0)

    def _step(i, x, first=False):
        z = (x + 2) % 3
        _gwait(x)
        _scale(i, rows[x], CHUNK)
        _wissue(x)
        if not first:
            _wwait(z)

        @pl.when(i + 2 < NFULL)
        def _():
            _fetch(i + 2, z)
            _gissue(i + 2, z)

    _fetch(0, 0)
    _gissue(0, 0)
    _fetch(1, 1)
    _gissue(1, 1)
    _step(0, 0, first=True)

    def _triple(t, _):
        i = 3 * t + 1
        _step(i, 1)
        _step(i + 1, 2)
        _step(i + 2, 0)
        return 0

    lax.fori_loop(0, (NFULL - 3) // 3, _triple, 0)
    _step(NFULL - 2, (NFULL - 2) % 3)
    _step(NFULL - 1, (NFULL - 1) % 3)
    _wwait((NFULL - 1) % 3)

    toff = NFULL * CHUNK
    pltpu.sync_copy(dst_hbm.at[pl.ds(ebase + toff, TAIL)], dst_t)
    pltpu.async_copy(h_hbm.at[srcs_v.at[pl.ds(toff, TAIL)]], rows_t,
                     sg0).wait()
    w16 = ws_v[pl.ds(toff, 16)]
    for l in range(TAIL):
        wb = _lane_bcast(w16, l)
        for v in range(D // 16):
            rsl = pl.ds(16 * v, 16)
            rows_t[l, rsl] = rows_t[l, rsl] * wb
    pltpu.sync_copy(rows_t, acc.at[dst_t], add=True)
    plsc.subcore_barrier()

    @pl.when(c == 0)
    def _():
        pltpu.sync_copy(acc.at[pl.ds(s * ROWS_PT, ROWS_PT)],
                        out0.at[pl.ds(s * ROWS_PT, ROWS_PT)])

    @pl.when(c == 1)
    def _():
        pltpu.sync_copy(acc.at[pl.ds(s * ROWS_PT, ROWS_PT)],
                        out1.at[pl.ds(s * ROWS_PT, ROWS_PT)])


def _combine_body(p0_ref, p1_ref, s_ref, h_ref, so_ref):
    h = p0_ref[...] + p1_ref[...]
    h_ref[...] = h
    so_ref[...] = s_ref[...] + h


_CROWS = 1024


def _combine(p0, p1, s_in):
    spec = pl.BlockSpec((_CROWS, D), lambda i: (i, 0))
    return pl.pallas_call(
        _combine_body,
        grid=(N_PAD // _CROWS,),
        in_specs=[spec, spec, spec],
        out_specs=[spec, spec],
        out_shape=[
            jax.ShapeDtypeStruct((N_PAD, D), jnp.float32),
            jax.ShapeDtypeStruct((N_PAD, D), jnp.float32),
        ],
    )(p0, p1, s_in)


@functools.partial(
    pl.kernel,
    out_type=jax.ShapeDtypeStruct((BATCH_N,), jnp.float32),
    mesh=_mesh,
    scratch_types=[
        pltpu.VMEM((BPW,), jnp.int32),
        pltpu.VMEM((BPW,), jnp.int32),
        pltpu.VMEM((BPW,), jnp.int32),
        pltpu.VMEM((BPW, D), jnp.float32),
        pltpu.VMEM((BPW, D), jnp.float32),
        pltpu.VMEM((BPW, D), jnp.float32),
        pltpu.VMEM((BPW, D), jnp.float32),
        pltpu.VMEM((BPW, D), jnp.float32),
        pltpu.VMEM((BPW, D), jnp.float32),
        pltpu.VMEM((BPW,), jnp.float32),
        pltpu.SemaphoreType.DMA,
    ],
)
def _gamma3(s2_hbm, p0_hbm, p1_hbm, users_hbm, items_hbm, out_hbm,
            u_v, it_v, ii_v, us_v, up0_v, up1_v, is_v, ip0_v, ip1_v,
            out_v, sem):
    wid = lax.axis_index("c") * NS + lax.axis_index("s")
    off = wid * BPW
    pltpu.sync_copy(users_hbm.at[pl.ds(off, BPW)], u_v)
    pltpu.sync_copy(items_hbm.at[pl.ds(off, BPW)], it_v)

    def _shift(i, _):
        sl = pl.ds(i * 16, 16)
        ii_v[sl] = it_v[sl] + N_USERS
        return 0

    lax.fori_loop(0, BPW // 16, _shift, 0)
    pltpu.async_copy(s2_hbm.at[u_v], us_v, sem)
    pltpu.async_copy(p0_hbm.at[u_v], up0_v, sem)
    pltpu.async_copy(p1_hbm.at[u_v], up1_v, sem)
    pltpu.async_copy(s2_hbm.at[ii_v], is_v, sem)
    pltpu.async_copy(p0_hbm.at[ii_v], ip0_v, sem)
    pltpu.async_copy(p1_hbm.at[ii_v], ip1_v, sem)
    for buf, hbm in ((us_v, s2_hbm), (up0_v, p0_hbm), (up1_v, p1_hbm),
                     (is_v, s2_hbm), (ip0_v, p0_hbm), (ip1_v, p1_hbm)):
        pltpu.make_async_copy(hbm.at[pl.ds(0, BPW)], buf, sem).wait()

    lane = lax.iota(jnp.int32, 16)

    def _group(g, _):
        def _one(l, accv):
            b = g * 16 + l
            acc = jnp.zeros((16,), jnp.float32)
            for v in range(D // 16):
                sl = pl.ds(16 * v, 16)
                urow = us_v[b, sl] + up0_v[b, sl] + up1_v[b, sl]
                irow = is_v[b, sl] + ip0_v[b, sl] + ip1_v[b, sl]
                acc = acc + urow * irow
            gvec = _lane_sum(acc) * (1.0 / 16.0)
            return jnp.where(lane == l, gvec, accv)

        vec = lax.fori_loop(0, 16, _one, jnp.zeros((16,), jnp.float32))
        out_v[pl.ds(g * 16, 16)] = vec
        return 0

    lax.fori_loop(0, BPW // 16, _group, 0)
    pltpu.sync_copy(out_v, out_hbm.at[pl.ds(off, BPW)])


def kernel(user_emb, item_emb, edge_index, edge_weight, users, items):
    all_emb = jnp.concatenate([user_emb, item_emb], axis=0)
    src = edge_index[0].astype(jnp.int32)
    dst = edge_index[1].astype(jnp.int32)
    users = users.astype(jnp.int32)
    items = items.astype(jnp.int32)

    pad = jnp.zeros((N_PAD - N_NODES, D), jnp.float32)
    h = jnp.concatenate([all_emb, pad], axis=0)
    ssum = h
    for _ in range(2):
        p0, p1 = _prop(h, src, dst, edge_weight)
        h, ssum = _combine(p0, p1, ssum)
    p0, p1 = _prop(h, src, dst, edge_weight)
    return _gamma3(ssum, p0, p1, users, items)
